# baseline (speedup 1.0000x reference)
import functools

import jax
import jax.numpy as jnp
from jax.experimental import pallas as pl

_BQ = 4096


def _nearest_proto_block(x_ref, p_ref, o_ref, *, n_out: int):
    x = x_ref[...]
    p = p_ref[...]
    x2 = jnp.sum(x * x, axis=1, keepdims=True)
    p2 = jnp.sum(p * p, axis=1)[None, :]
    dot = jax.lax.dot_general(
        x, p, (((1,), (1,)), ((), ())),
        preferred_element_type=jnp.float32)
    d2 = x2 + p2 - 2.0 * dot
    lab = jnp.argmin(d2, axis=1).astype(jnp.int32) + 1
    cols = jax.lax.broadcasted_iota(jnp.int32, (x.shape[0], n_out), 1)
    o_ref[...] = (cols == lab[:, None]).astype(jnp.float32)


def kernel(x, protos):
    q, d = x.shape
    k, _ = protos.shape
    n_out = k + 1
    return pl.pallas_call(
        functools.partial(_nearest_proto_block, n_out=n_out),
        grid=(q // _BQ,),
        in_specs=[
            pl.BlockSpec((_BQ, d), lambda i: (i, 0)),
            pl.BlockSpec((k, d), lambda i: (0, 0)),
        ],
        out_specs=pl.BlockSpec((_BQ, n_out), lambda i: (i, 0)),
        out_shape=jax.ShapeDtypeStruct((q, n_out), jnp.float32),
    )(x, protos)


# speedup vs baseline: 1.0052x; 1.0052x over previous
import functools

import jax
import jax.numpy as jnp
from jax.experimental import pallas as pl
from jax.experimental.pallas import tpu as pltpu

_BQ = 2048


def _nearest_proto_block(x_ref, p_ref, o_ref, *, n_out: int):
    x = x_ref[...]
    p = p_ref[...]
    x2 = jnp.sum(x * x, axis=1, keepdims=True)
    p2 = jnp.sum(p * p, axis=1)[None, :]
    dot = jax.lax.dot_general(
        x, p, (((1,), (1,)), ((), ())),
        preferred_element_type=jnp.float32)
    d2 = x2 + p2 - 2.0 * dot
    lab = jnp.argmin(d2, axis=1).astype(jnp.int32) + 1
    cols = jax.lax.broadcasted_iota(jnp.int32, (x.shape[0], n_out), 1)
    o_ref[...] = (cols == lab[:, None]).astype(jnp.float32)


def kernel(x, protos):
    q, d = x.shape
    k, _ = protos.shape
    n_out = k + 1
    return pl.pallas_call(
        functools.partial(_nearest_proto_block, n_out=n_out),
        grid=(q // _BQ,),
        in_specs=[
            pl.BlockSpec((_BQ, d), lambda i: (i, 0)),
            pl.BlockSpec((k, d), lambda i: (0, 0)),
        ],
        out_specs=pl.BlockSpec((_BQ, n_out), lambda i: (i, 0)),
        out_shape=jax.ShapeDtypeStruct((q, n_out), jnp.float32),
        compiler_params=pltpu.CompilerParams(
            dimension_semantics=("parallel",)),
    )(x, protos)
